# baseline (speedup 1.0000x reference)
import functools

import jax
import jax.numpy as jnp
from jax import lax
from jax.experimental import pallas as pl
from jax.experimental.pallas import tpu as pltpu
from jax.experimental.pallas import tpu_sc as plsc

D = 64
V = 1024

_SCORE_PREC = lax.Precision.DEFAULT
_MM_PREC = lax.Precision.DEFAULT


_CHL = 128


def _argmin_body(ng, w_ref, rs_ref, cbt_ref, idx_ref):
    x = w_ref[...] / rs_ref[...]
    cbt = cbt_ref[...]
    c2h = 0.5 * jnp.sum(cbt * cbt, axis=0, keepdims=True)
    br = x.shape[0]
    iota = lax.broadcasted_iota(jnp.int32, (br, V), 1).astype(jnp.float32)
    cols = []
    for g in range(ng):
        xg = x[:, D * g:D * (g + 1)]
        gmat = lax.dot_general(xg, cbt, (((1,), (0,)), ((), ())),
                               precision=_SCORE_PREC)
        t = c2h - gmat
        m = jnp.min(t, axis=1, keepdims=True)
        first = jnp.min(jnp.where(t == m, iota, jnp.float32(V)), axis=1)
        cols.append(first.astype(jnp.int32))
    idx_ref[...] = jnp.stack(cols, axis=1)


def _argmin_gather_body(ng, w_ref, rs_ref, cbt_ref, q_ref):
    x = w_ref[...] / rs_ref[...]
    cbt = cbt_ref[...]
    c2h = 0.5 * jnp.sum(cbt * cbt, axis=0, keepdims=True)
    br = x.shape[0]
    iota = lax.broadcasted_iota(jnp.int32, (br, V), 1).astype(jnp.float32)
    qs = []
    for g in range(ng):
        xg = x[:, D * g:D * (g + 1)]
        gmat = lax.dot_general(xg, cbt, (((1,), (0,)), ((), ())),
                               precision=_SCORE_PREC)
        t = c2h - gmat
        m = jnp.min(t, axis=1, keepdims=True)
        first = jnp.min(jnp.where(t == m, iota, jnp.float32(V)), axis=1,
                        keepdims=True)
        oh = (iota == first).astype(jnp.float32)
        qs.append(lax.dot_general(oh, cbt, (((1,), (1,)), ((), ())),
                                  precision=_MM_PREC))
    q_ref[...] = jnp.concatenate(qs, axis=1)


def _pq_argmin_gather(w, rs, cbt, block_rows):
    n, width = w.shape
    ng = width // D
    grid = (n // block_rows,)
    return pl.pallas_call(
        functools.partial(_argmin_gather_body, ng),
        grid=grid,
        in_specs=[
            pl.BlockSpec((block_rows, width), lambda i: (i, 0)),
            pl.BlockSpec((block_rows, 1), lambda i: (i, 0)),
            pl.BlockSpec((D, V), lambda i: (0, 0)),
        ],
        out_specs=pl.BlockSpec((block_rows, width), lambda i: (i, 0)),
        out_shape=jax.ShapeDtypeStruct((n, width), jnp.float32),
    )(w, rs, cbt)


def _pq_argmin(w, rs, cbt, block_rows):
    n, width = w.shape
    ng = width // D
    grid = (n // block_rows,)
    return pl.pallas_call(
        functools.partial(_argmin_body, ng),
        grid=grid,
        in_specs=[
            pl.BlockSpec((block_rows, width), lambda i: (i, 0)),
            pl.BlockSpec((block_rows, 1), lambda i: (i, 0)),
            pl.BlockSpec((D, V), lambda i: (0, 0)),
        ],
        out_specs=pl.BlockSpec((block_rows, ng), lambda i: (i, 0)),
        out_shape=jax.ShapeDtypeStruct((n, ng), jnp.int32),
    )(w, rs, cbt)


_CH = 128
_NBUF = 2


def _sc_gather(table, idx):
    n = idx.shape[0]
    nw = 32
    b_per_w = n // nw
    n_ch = b_per_w // _CH
    nbuf = min(_NBUF, n_ch)
    chw = _CH * D
    mesh = plsc.VectorSubcoreMesh(core_axis_name="c", subcore_axis_name="s")

    @functools.partial(
        pl.kernel,
        out_type=jax.ShapeDtypeStruct((n * D,), jnp.float32),
        mesh=mesh,
        compiler_params=pltpu.CompilerParams(use_tc_tiling_on_sc=False,
                                             needs_layout_passes=False),
        scratch_types=[
            pltpu.VMEM((V * D,), jnp.float32),
            pltpu.VMEM((b_per_w,), jnp.int32),
            pltpu.VMEM((_NBUF * _CH * D,), jnp.float32),
            pltpu.SemaphoreType.DMA,
        ],
    )
    def k(table_hbm, idx_hbm, out_hbm, table_v, idx_v, obuf, osem):
        wid = lax.axis_index("s") * 2 + lax.axis_index("c")
        base = wid * b_per_w
        pltpu.sync_copy(table_hbm, table_v)
        pltpu.sync_copy(idx_hbm.at[pl.ds(base, b_per_w)], idx_v)
        iota16 = lax.broadcasted_iota(jnp.int32, (16,), 0)

        def chunk(j, carry):
            b = lax.rem(j, nbuf)

            @pl.when(j >= nbuf)
            def _():
                pltpu.make_async_copy(
                    table_hbm.at[pl.ds(0, chw)],
                    obuf.at[pl.ds(0, chw)],
                    osem,
                ).wait()

            for t in range(_CH // 16):
                row16 = idx_v[pl.ds(j * _CH + t * 16, 16)]
                rowbase16 = row16 * D
                obase16 = (b * _CH + t * 16 + iota16) * D
                for c in range(D):
                    v = plsc.load_gather(table_v, [rowbase16 + c])
                    plsc.store_scatter(obuf, [obase16 + c], v)
            pltpu.async_copy(
                obuf.at[pl.ds(b * chw, chw)],
                out_hbm.at[pl.ds((base + j * _CH) * D, chw)],
                osem,
            )
            return carry

        lax.fori_loop(0, n_ch, chunk, 0)
        for _ in range(nbuf):
            pltpu.make_async_copy(
                table_hbm.at[pl.ds(0, chw)],
                obuf.at[pl.ds(0, chw)],
                osem,
            ).wait()

    return k(table.reshape(-1), idx).reshape(n, D)


def _mm_body(uq_ref, rsu_ref, bq_ref, rsb_ref, out_ref):
    a = uq_ref[...] * rsu_ref[...]
    b = bq_ref[...] * rsb_ref[...]
    out_ref[...] = lax.dot_general(a, b, (((1,), (0,)), ((), ())),
                                   precision=_MM_PREC)


def _scaled_matmul(uq, rs_u, bq, rs_b, block_rows):
    n, k = uq.shape
    m = bq.shape[1]
    grid = (n // block_rows,)
    return pl.pallas_call(
        _mm_body,
        grid=grid,
        in_specs=[
            pl.BlockSpec((block_rows, k), lambda i: (i, 0)),
            pl.BlockSpec((block_rows, 1), lambda i: (i, 0)),
            pl.BlockSpec((k, m), lambda i: (0, 0)),
            pl.BlockSpec((k, 1), lambda i: (0, 0)),
        ],
        out_specs=pl.BlockSpec((block_rows, m), lambda i: (i, 0)),
        out_shape=jax.ShapeDtypeStruct((n, m), jnp.float32),
    )(uq, rs_u, bq, rs_b)


def kernel(U, B, rs_U, rs_B, cb_U, cb_B):
    cbt_U = cb_U.T
    cbt_B = cb_B.T
    Bq = _pq_argmin_gather(B, rs_B, cbt_B, block_rows=256)
    idx_U = _pq_argmin(U, rs_U, cbt_U, block_rows=512)
    qU = _sc_gather(cb_U, idx_U.reshape(-1))
    Uq = qU.reshape(16384, 256)
    return _scaled_matmul(Uq, rs_U, Bq, rs_B, block_rows=1024)


# speedup vs baseline: 1.5130x; 1.5130x over previous
import functools

import jax
import jax.numpy as jnp
from jax import lax
from jax.experimental import pallas as pl
from jax.experimental.pallas import tpu as pltpu
from jax.experimental.pallas import tpu_sc as plsc

D = 64
V = 1024

_SCORE_PREC = lax.Precision.DEFAULT
_MM_PREC = lax.Precision.DEFAULT


_CHL = 128


def _argmin_body(ng, w_ref, rs_ref, cbt_ref, idx_ref):
    x = w_ref[...] / rs_ref[...]
    cbt = cbt_ref[...]
    c2h = 0.5 * jnp.sum(cbt * cbt, axis=0, keepdims=True)
    br = x.shape[0]
    iota = lax.broadcasted_iota(jnp.int32, (br, V), 1).astype(jnp.float32)
    cols = []
    for g in range(ng):
        xg = x[:, D * g:D * (g + 1)]
        gmat = lax.dot_general(xg, cbt, (((1,), (0,)), ((), ())),
                               precision=_SCORE_PREC)
        t = c2h - gmat
        m = jnp.min(t, axis=1, keepdims=True)
        first = jnp.min(jnp.where(t == m, iota, jnp.float32(V)), axis=1)
        cols.append(first.astype(jnp.int32))
    idx_ref[...] = jnp.stack(cols, axis=1)


def _argmin_gather_body(ng, w_ref, rs_ref, cbt_ref, q_ref):
    x = w_ref[...] / rs_ref[...]
    cbt = cbt_ref[...]
    c2h = 0.5 * jnp.sum(cbt * cbt, axis=0, keepdims=True)
    br = x.shape[0]
    iota = lax.broadcasted_iota(jnp.int32, (br, V), 1).astype(jnp.float32)
    qs = []
    for g in range(ng):
        xg = x[:, D * g:D * (g + 1)]
        gmat = lax.dot_general(xg, cbt, (((1,), (0,)), ((), ())),
                               precision=_SCORE_PREC)
        t = c2h - gmat
        m = jnp.min(t, axis=1, keepdims=True)
        first = jnp.min(jnp.where(t == m, iota, jnp.float32(V)), axis=1,
                        keepdims=True)
        oh = (iota == first).astype(jnp.float32)
        qs.append(lax.dot_general(oh, cbt, (((1,), (1,)), ((), ())),
                                  precision=_MM_PREC))
    q_ref[...] = jnp.concatenate(qs, axis=1)


def _pq_argmin_gather(w, rs, cbt, block_rows):
    n, width = w.shape
    ng = width // D
    grid = (n // block_rows,)
    return pl.pallas_call(
        functools.partial(_argmin_gather_body, ng),
        grid=grid,
        in_specs=[
            pl.BlockSpec((block_rows, width), lambda i: (i, 0)),
            pl.BlockSpec((block_rows, 1), lambda i: (i, 0)),
            pl.BlockSpec((D, V), lambda i: (0, 0)),
        ],
        out_specs=pl.BlockSpec((block_rows, width), lambda i: (i, 0)),
        out_shape=jax.ShapeDtypeStruct((n, width), jnp.float32),
    )(w, rs, cbt)


def _pq_argmin(w, rs, cbt, block_rows, row_lo=0, row_hi=None):
    n, width = w.shape
    row_hi = n if row_hi is None else row_hi
    ng = width // D
    nrows = row_hi - row_lo
    off = row_lo // block_rows
    grid = (nrows // block_rows,)
    return pl.pallas_call(
        functools.partial(_argmin_body, ng),
        grid=grid,
        in_specs=[
            pl.BlockSpec((block_rows, width), lambda i: (i + off, 0)),
            pl.BlockSpec((block_rows, 1), lambda i: (i + off, 0)),
            pl.BlockSpec((D, V), lambda i: (0, 0)),
        ],
        out_specs=pl.BlockSpec((block_rows, ng), lambda i: (i, 0)),
        out_shape=jax.ShapeDtypeStruct((nrows, ng), jnp.int32),
    )(w, rs, cbt)


_CH = 128


_NBUF = 8


def _sc_gather(table, idx):
    n = idx.shape[0]
    nw = 32
    b_per_w = n // nw
    n_ch = b_per_w // _CH
    nbuf = min(_NBUF, n_ch)
    mesh = plsc.VectorSubcoreMesh(core_axis_name="c", subcore_axis_name="s")

    @functools.partial(
        pl.kernel,
        out_type=jax.ShapeDtypeStruct((n, D), jnp.float32),
        mesh=mesh,
        compiler_params=pltpu.CompilerParams(use_tc_tiling_on_sc=False),
        scratch_types=[
            pltpu.VMEM((b_per_w,), jnp.int32),
            pltpu.VMEM((nbuf, _CH, D), jnp.float32),
            pltpu.SemaphoreType.DMA,
            pltpu.SemaphoreType.DMA,
            pltpu.SemaphoreType.DMA,
        ],
    )
    def k(table_hbm, idx_hbm, out_hbm, idx_v, bufs, gsem_a, gsem_b, osem):
        wid = lax.axis_index("s") * 2 + lax.axis_index("c")
        base = wid * b_per_w
        pltpu.sync_copy(idx_hbm.at[pl.ds(base, b_per_w)], idx_v)

        def fire_gather(j):
            return pltpu.async_copy(
                table_hbm.at[idx_v.at[pl.ds(j * _CH, _CH)]],
                bufs.at[j % nbuf],
                gsem_a if j % 2 == 0 else gsem_b,
            )

        ahead = max(1, nbuf - 2)
        gathers = [None] * n_ch
        outs = [None] * n_ch
        for j in range(min(ahead, n_ch)):
            gathers[j] = fire_gather(j)
        for j in range(n_ch):
            gp = j + ahead
            if gp < n_ch:
                if gp - nbuf >= 0:
                    outs[gp - nbuf].wait()
                gathers[gp] = fire_gather(gp)
            gathers[j].wait()
            outs[j] = pltpu.async_copy(
                bufs.at[j % nbuf],
                out_hbm.at[pl.ds(base + j * _CH, _CH)],
                osem,
            )
        for j in range(max(0, n_ch - nbuf), n_ch):
            outs[j].wait()

    return k(table, idx)


def _mm_body(uq_ref, rsu_ref, bq_ref, rsb_ref, out_ref):
    a = uq_ref[...] * rsu_ref[...]
    b = bq_ref[...] * rsb_ref[...]
    out_ref[...] = lax.dot_general(a, b, (((1,), (0,)), ((), ())),
                                   precision=_MM_PREC)


def _scaled_matmul(uq, rs_u, bq, rs_b, block_rows):
    n, k = uq.shape
    m = bq.shape[1]
    grid = (n // block_rows,)
    return pl.pallas_call(
        _mm_body,
        grid=grid,
        in_specs=[
            pl.BlockSpec((block_rows, k), lambda i: (i, 0)),
            pl.BlockSpec((block_rows, 1), lambda i: (i, 0)),
            pl.BlockSpec((k, m), lambda i: (0, 0)),
            pl.BlockSpec((k, 1), lambda i: (0, 0)),
        ],
        out_specs=pl.BlockSpec((block_rows, m), lambda i: (i, 0)),
        out_shape=jax.ShapeDtypeStruct((n, m), jnp.float32),
    )(uq, rs_u, bq, rs_b)


def kernel(U, B, rs_U, rs_B, cb_U, cb_B):
    cbt_U = cb_U.T
    cbt_B = cb_B.T
    Bq = _pq_argmin_gather(B, rs_B, cbt_B, block_rows=256)
    half = 8192
    idx_U0 = _pq_argmin(U, rs_U, cbt_U, 512, 0, half)
    qU0 = _sc_gather(cb_U, idx_U0.reshape(-1))
    idx_U1 = _pq_argmin(U, rs_U, cbt_U, 512, half, 16384)
    qU1 = _sc_gather(cb_U, idx_U1.reshape(-1))
    Uq = jnp.concatenate(
        [qU0.reshape(half, 256), qU1.reshape(half, 256)], axis=0)
    return _scaled_matmul(Uq, rs_U, Bq, rs_B, block_rows=1024)


# speedup vs baseline: 1.5813x; 1.0451x over previous
import functools

import jax
import jax.numpy as jnp
from jax import lax
from jax.experimental import pallas as pl
from jax.experimental.pallas import tpu as pltpu
from jax.experimental.pallas import tpu_sc as plsc

D = 64
V = 1024

_SCORE_PREC = lax.Precision.DEFAULT
_MM_PREC = lax.Precision.DEFAULT


_CHL = 128


def _argmin_body(ng, w_ref, rs_ref, cbt_ref, idx_ref):
    x = w_ref[...] / rs_ref[...]
    cbt = cbt_ref[...]
    c2h = 0.5 * jnp.sum(cbt * cbt, axis=0, keepdims=True)
    h1 = c2h.astype(jnp.bfloat16).astype(jnp.float32)
    h2 = (c2h - h1).astype(jnp.bfloat16).astype(jnp.float32)
    h3 = c2h - h1 - h2
    cbt_aug = jnp.concatenate([cbt, h1, h2, h3], axis=0)
    br = x.shape[0]
    ones3 = jnp.full((br, 3), -1.0, jnp.float32)
    iota = lax.broadcasted_iota(jnp.int32, (br, V), 1).astype(jnp.float32)
    cols = []
    for g in range(ng):
        xg = jnp.concatenate([x[:, D * g:D * (g + 1)], ones3], axis=1)
        t = lax.dot_general(xg, cbt_aug, (((1,), (0,)), ((), ())),
                            precision=_SCORE_PREC)
        m = jnp.max(t, axis=1, keepdims=True)
        first = jnp.min(jnp.where(t == m, iota, jnp.float32(V)), axis=1)
        cols.append(first.astype(jnp.int32))
    idx_ref[...] = jnp.stack(cols, axis=1)


def _argmin_gather_body(ng, w_ref, rs_ref, cbt_ref, q_ref):
    x = w_ref[...] / rs_ref[...]
    cbt = cbt_ref[...]
    c2h = 0.5 * jnp.sum(cbt * cbt, axis=0, keepdims=True)
    br = x.shape[0]
    iota = lax.broadcasted_iota(jnp.int32, (br, V), 1).astype(jnp.float32)
    qs = []
    for g in range(ng):
        xg = x[:, D * g:D * (g + 1)]
        gmat = lax.dot_general(xg, cbt, (((1,), (0,)), ((), ())),
                               precision=_SCORE_PREC)
        t = c2h - gmat
        m = jnp.min(t, axis=1, keepdims=True)
        first = jnp.min(jnp.where(t == m, iota, jnp.float32(V)), axis=1,
                        keepdims=True)
        oh = (iota == first).astype(jnp.float32)
        qs.append(lax.dot_general(oh, cbt, (((1,), (1,)), ((), ())),
                                  precision=_MM_PREC))
    q_ref[...] = jnp.concatenate(qs, axis=1)


def _pq_argmin_gather(w, rs, cbt, block_rows):
    n, width = w.shape
    ng = width // D
    grid = (n // block_rows,)
    return pl.pallas_call(
        functools.partial(_argmin_gather_body, ng),
        grid=grid,
        in_specs=[
            pl.BlockSpec((block_rows, width), lambda i: (i, 0)),
            pl.BlockSpec((block_rows, 1), lambda i: (i, 0)),
            pl.BlockSpec((D, V), lambda i: (0, 0)),
        ],
        out_specs=pl.BlockSpec((block_rows, width), lambda i: (i, 0)),
        out_shape=jax.ShapeDtypeStruct((n, width), jnp.float32),
    )(w, rs, cbt)


def _pq_argmin(w, rs, cbt, block_rows, row_lo=0, row_hi=None):
    n, width = w.shape
    row_hi = n if row_hi is None else row_hi
    ng = width // D
    nrows = row_hi - row_lo
    off = row_lo // block_rows
    grid = (nrows // block_rows,)
    return pl.pallas_call(
        functools.partial(_argmin_body, ng),
        grid=grid,
        in_specs=[
            pl.BlockSpec((block_rows, width), lambda i: (i + off, 0)),
            pl.BlockSpec((block_rows, 1), lambda i: (i + off, 0)),
            pl.BlockSpec((D, V), lambda i: (0, 0)),
        ],
        out_specs=pl.BlockSpec((block_rows, ng), lambda i: (i, 0)),
        out_shape=jax.ShapeDtypeStruct((nrows, ng), jnp.int32),
    )(w, rs, cbt)


_CH = 128


_NBUF = 8


def _sc_gather(table, idx):
    n = idx.shape[0]
    nw = 32
    b_per_w = n // nw
    n_ch = b_per_w // _CH
    nbuf = min(_NBUF, n_ch)
    mesh = plsc.VectorSubcoreMesh(core_axis_name="c", subcore_axis_name="s")

    @functools.partial(
        pl.kernel,
        out_type=jax.ShapeDtypeStruct((n, D), jnp.float32),
        mesh=mesh,
        compiler_params=pltpu.CompilerParams(use_tc_tiling_on_sc=False),
        scratch_types=[
            pltpu.VMEM((b_per_w,), jnp.int32),
            pltpu.VMEM((nbuf, _CH, D), jnp.float32),
            pltpu.SemaphoreType.DMA,
            pltpu.SemaphoreType.DMA,
            pltpu.SemaphoreType.DMA,
        ],
    )
    def k(table_hbm, idx_hbm, out_hbm, idx_v, bufs, gsem_a, gsem_b, osem):
        wid = lax.axis_index("s") * 2 + lax.axis_index("c")
        base = wid * b_per_w
        pltpu.sync_copy(idx_hbm.at[pl.ds(base, b_per_w)], idx_v)

        def fire_gather(j):
            return pltpu.async_copy(
                table_hbm.at[idx_v.at[pl.ds(j * _CH, _CH)]],
                bufs.at[j % nbuf],
                gsem_a if j % 2 == 0 else gsem_b,
            )

        ahead = max(1, nbuf - 2)
        gathers = [None] * n_ch
        outs = [None] * n_ch
        for j in range(min(ahead, n_ch)):
            gathers[j] = fire_gather(j)
        for j in range(n_ch):
            gp = j + ahead
            if gp < n_ch:
                if gp - nbuf >= 0:
                    outs[gp - nbuf].wait()
                gathers[gp] = fire_gather(gp)
            gathers[j].wait()
            outs[j] = pltpu.async_copy(
                bufs.at[j % nbuf],
                out_hbm.at[pl.ds(base + j * _CH, _CH)],
                osem,
            )
        for j in range(max(0, n_ch - nbuf), n_ch):
            outs[j].wait()

    return k(table, idx)


def _mm_body(uq_ref, rsu_ref, bq_ref, rsb_ref, out_ref):
    a = uq_ref[...] * rsu_ref[...]
    b = bq_ref[...] * rsb_ref[...]
    out_ref[...] = lax.dot_general(a, b, (((1,), (0,)), ((), ())),
                                   precision=_MM_PREC)


def _scaled_matmul(uq, rs_u, bq, rs_b, block_rows):
    n, k = uq.shape
    m = bq.shape[1]
    grid = (n // block_rows,)
    return pl.pallas_call(
        _mm_body,
        grid=grid,
        in_specs=[
            pl.BlockSpec((block_rows, k), lambda i: (i, 0)),
            pl.BlockSpec((block_rows, 1), lambda i: (i, 0)),
            pl.BlockSpec((k, m), lambda i: (0, 0)),
            pl.BlockSpec((k, 1), lambda i: (0, 0)),
        ],
        out_specs=pl.BlockSpec((block_rows, m), lambda i: (i, 0)),
        out_shape=jax.ShapeDtypeStruct((n, m), jnp.float32),
    )(uq, rs_u, bq, rs_b)


def kernel(U, B, rs_U, rs_B, cb_U, cb_B):
    cbt_U = cb_U.T
    cbt_B = cb_B.T
    Bq = _pq_argmin_gather(B, rs_B, cbt_B, block_rows=256)
    idx_U = _pq_argmin(U, rs_U, cbt_U, 512)
    qU = _sc_gather(cb_U, idx_U.reshape(-1))
    Uq = qU.reshape(16384, 256)
    return _scaled_matmul(Uq, rs_U, Bq, rs_B, block_rows=1024)


# speedup vs baseline: 1.6433x; 1.0392x over previous
import functools

import jax
import jax.numpy as jnp
from jax import lax
from jax.experimental import pallas as pl
from jax.experimental.pallas import tpu as pltpu
from jax.experimental.pallas import tpu_sc as plsc

D = 64
V = 1024

_SCORE_PREC = lax.Precision.DEFAULT
_MM_PREC = lax.Precision.DEFAULT


_CHL = 128


def _argmin_body(ng, w_ref, rs_ref, cbt_ref, idx_ref):
    x = w_ref[...] / rs_ref[...]
    cbt = cbt_ref[...]
    c2h = 0.5 * jnp.sum(cbt * cbt, axis=0, keepdims=True)
    h1 = c2h.astype(jnp.bfloat16).astype(jnp.float32)
    h2 = (c2h - h1).astype(jnp.bfloat16).astype(jnp.float32)
    h3 = c2h - h1 - h2
    cbt_aug = jnp.concatenate([cbt, h1, h2, h3], axis=0)
    br = x.shape[0]
    ones3 = jnp.full((br, 3), -1.0, jnp.float32)
    iota = lax.broadcasted_iota(jnp.int32, (br, V), 1).astype(jnp.float32)
    cols = []
    for g in range(ng):
        xg = jnp.concatenate([x[:, D * g:D * (g + 1)], ones3], axis=1)
        t = lax.dot_general(xg, cbt_aug, (((1,), (0,)), ((), ())),
                            precision=_SCORE_PREC)
        m = jnp.max(t, axis=1, keepdims=True)
        first = jnp.min(jnp.where(t == m, iota, jnp.float32(V)), axis=1)
        cols.append(first.astype(jnp.int32))
    idx_ref[...] = jnp.stack(cols, axis=1)


def _argmin_gather_body(ng, w_ref, rs_ref, cbt_ref, q_ref):
    x = w_ref[...] / rs_ref[...]
    cbt = cbt_ref[...]
    c2h = 0.5 * jnp.sum(cbt * cbt, axis=0, keepdims=True)
    br = x.shape[0]
    iota = lax.broadcasted_iota(jnp.int32, (br, V), 1).astype(jnp.float32)
    qs = []
    for g in range(ng):
        xg = x[:, D * g:D * (g + 1)]
        gmat = lax.dot_general(xg, cbt, (((1,), (0,)), ((), ())),
                               precision=_SCORE_PREC)
        t = c2h - gmat
        m = jnp.min(t, axis=1, keepdims=True)
        first = jnp.min(jnp.where(t == m, iota, jnp.float32(V)), axis=1,
                        keepdims=True)
        oh = (iota == first).astype(jnp.float32)
        qs.append(lax.dot_general(oh, cbt, (((1,), (1,)), ((), ())),
                                  precision=_MM_PREC))
    q_ref[...] = jnp.concatenate(qs, axis=1)


def _pq_argmin_gather(w, rs, cbt, block_rows):
    n, width = w.shape
    ng = width // D
    grid = (n // block_rows,)
    return pl.pallas_call(
        functools.partial(_argmin_gather_body, ng),
        grid=grid,
        in_specs=[
            pl.BlockSpec((block_rows, width), lambda i: (i, 0)),
            pl.BlockSpec((block_rows, 1), lambda i: (i, 0)),
            pl.BlockSpec((D, V), lambda i: (0, 0)),
        ],
        out_specs=pl.BlockSpec((block_rows, width), lambda i: (i, 0)),
        out_shape=jax.ShapeDtypeStruct((n, width), jnp.float32),
    )(w, rs, cbt)


def _pq_argmin(w, rs, cbt, block_rows, row_lo=0, row_hi=None):
    n, width = w.shape
    row_hi = n if row_hi is None else row_hi
    ng = width // D
    nrows = row_hi - row_lo
    off = row_lo // block_rows
    grid = (nrows // block_rows,)
    return pl.pallas_call(
        functools.partial(_argmin_body, ng),
        grid=grid,
        in_specs=[
            pl.BlockSpec((block_rows, width), lambda i: (i + off, 0)),
            pl.BlockSpec((block_rows, 1), lambda i: (i + off, 0)),
            pl.BlockSpec((D, V), lambda i: (0, 0)),
        ],
        out_specs=pl.BlockSpec((block_rows, ng), lambda i: (i, 0)),
        out_shape=jax.ShapeDtypeStruct((nrows, ng), jnp.int32),
    )(w, rs, cbt)


_CH = 128


_NBUF = 8


def _sc_gather(table, idx):
    n = idx.shape[0]
    nw = 32
    b_per_w = n // nw
    n_ch = b_per_w // _CH
    nbuf = min(_NBUF, n_ch)
    mesh = plsc.VectorSubcoreMesh(core_axis_name="c", subcore_axis_name="s")

    @functools.partial(
        pl.kernel,
        out_type=jax.ShapeDtypeStruct((n, D), jnp.float32),
        mesh=mesh,
        compiler_params=pltpu.CompilerParams(use_tc_tiling_on_sc=False),
        scratch_types=[
            pltpu.VMEM((b_per_w,), jnp.int32),
            pltpu.VMEM((nbuf, _CH, D), jnp.float32),
            pltpu.SemaphoreType.DMA,
            pltpu.SemaphoreType.DMA,
            pltpu.SemaphoreType.DMA,
        ],
    )
    def k(table_hbm, idx_hbm, out_hbm, idx_v, bufs, gsem_a, gsem_b, osem):
        wid = lax.axis_index("s") * 2 + lax.axis_index("c")
        base = wid * b_per_w
        pltpu.sync_copy(idx_hbm.at[pl.ds(base, b_per_w)], idx_v)

        def fire_gather(j):
            return pltpu.async_copy(
                table_hbm.at[idx_v.at[pl.ds(j * _CH, _CH)]],
                bufs.at[j % nbuf],
                gsem_a if j % 2 == 0 else gsem_b,
            )

        ahead = max(1, nbuf - 2)
        gathers = [None] * n_ch
        outs = [None] * n_ch
        for j in range(min(ahead, n_ch)):
            gathers[j] = fire_gather(j)
        for j in range(n_ch):
            gp = j + ahead
            if gp < n_ch:
                if gp - nbuf >= 0:
                    outs[gp - nbuf].wait()
                gathers[gp] = fire_gather(gp)
            gathers[j].wait()
            outs[j] = pltpu.async_copy(
                bufs.at[j % nbuf],
                out_hbm.at[pl.ds(base + j * _CH, _CH)],
                osem,
            )
        for j in range(max(0, n_ch - nbuf), n_ch):
            outs[j].wait()

    return k(table, idx)


def _mm_body(uq_ref, rsu_ref, bq_ref, rsb_ref, out_ref):
    a = uq_ref[...] * rsu_ref[...]
    b = bq_ref[...] * rsb_ref[...]
    out_ref[...] = lax.dot_general(a, b, (((1,), (0,)), ((), ())),
                                   precision=_MM_PREC)


def _scaled_matmul(uq, rs_u, bq, rs_b, block_rows):
    n, k = uq.shape
    m = bq.shape[1]
    grid = (n // block_rows,)
    return pl.pallas_call(
        _mm_body,
        grid=grid,
        in_specs=[
            pl.BlockSpec((block_rows, k), lambda i: (i, 0)),
            pl.BlockSpec((block_rows, 1), lambda i: (i, 0)),
            pl.BlockSpec((k, m), lambda i: (0, 0)),
            pl.BlockSpec((k, 1), lambda i: (0, 0)),
        ],
        out_specs=pl.BlockSpec((block_rows, m), lambda i: (i, 0)),
        out_shape=jax.ShapeDtypeStruct((n, m), jnp.float32),
    )(uq, rs_u, bq, rs_b)


def kernel(U, B, rs_U, rs_B, cb_U, cb_B):
    cbt_U = cb_U.T
    cbt_B = cb_B.T
    Bq = _pq_argmin_gather(B, rs_B, cbt_B, block_rows=256)
    idx_U = _pq_argmin(U, rs_U, cbt_U, 1024)
    qU = _sc_gather(cb_U, idx_U.reshape(-1))
    Uq = qU.reshape(16384, 256)
    return _scaled_matmul(Uq, rs_U, Bq, rs_B, block_rows=2048)
